# baseline (speedup 1.0000x reference)
import functools

import jax
import jax.numpy as jnp
from jax import lax
from jax.experimental import pallas as pl
from jax.experimental.pallas import tpu as pltpu
from jax.experimental.pallas import tpu_sc as plsc

NBUF = 3
LANES = 16
BT = 128


def _build(B, L, V, D, NC, NS):
    NW = NC * NS
    assert B // BT == NW
    EG = D // 8
    BLK = EG * 8 * BT
    LSTRIDE = EG * NW * 8 * BT

    mesh = plsc.VectorSubcoreMesh(
        core_axis_name="c", subcore_axis_name="s",
        num_cores=NC, num_subcores=NS)

    @functools.partial(
        pl.kernel,
        out_type=jax.ShapeDtypeStruct((L * LSTRIDE,), jnp.float32),
        mesh=mesh,
        scratch_types=[
            pltpu.VMEM((BT, L), jnp.int32),
            pltpu.VMEM((L, D), jnp.float32),
            pltpu.VMEM((NBUF, BT, D), jnp.float32),
            pltpu.VMEM((NBUF, BLK), jnp.float32),
            pltpu.VMEM((NBUF, BT), jnp.int32),
            pltpu.SemaphoreType.DMA,
            pltpu.SemaphoreType.DMA,
            pltpu.SemaphoreType.DMA,
            pltpu.SemaphoreType.DMA,
            pltpu.SemaphoreType.DMA,
            pltpu.SemaphoreType.DMA,
        ],
        compiler_params=pltpu.CompilerParams(
            use_tc_tiling_on_sc=False, needs_layout_passes=False),
    )
    def body(idx_hbm, table_hbm, pos_hbm, out_hbm,
             idx_v, pos_v, rows_v, blk_v, gidx_v, g0, g1, g2, o0, o1, o2):
        gsems = (g0, g1, g2)
        osems = (o0, o1, o2)
        wid = lax.axis_index("s") * NC + lax.axis_index("c")
        bat_base = wid * BT

        pltpu.sync_copy(pos_hbm, pos_v)
        pltpu.sync_copy(idx_hbm.at[pl.ds(bat_base, BT)], idx_v)

        it = lax.iota(jnp.int32, LANES)
        ivec0 = ((it >> 3) << 10) + ((it & 7) << 7)
        jvecs = [it + g * LANES for g in range(BT // LANES)]

        def fire_gather(c, b):
            for g in range(BT // LANES):
                toks = plsc.load_gather(
                    idx_v, [jvecs[g], jnp.full((LANES,), c, jnp.int32)])
                gidx_v[b, pl.ds(g * LANES, LANES)] = toks << 1
            pltpu.async_copy(
                table_hbm.at[gidx_v.at[b]], rows_v.at[b], gsems[b])

        def drain_gather(b):
            pltpu.make_async_copy(
                table_hbm.at[pl.ds(0, BT)], rows_v.at[b], gsems[b]).wait()

        def fire_store(c, b):
            base = c * LSTRIDE + wid * (8 * BT)
            for e in range(EG):
                pltpu.async_copy(
                    blk_v.at[b, pl.ds(e * 8 * BT, 8 * BT)],
                    out_hbm.at[pl.ds(base + e * (NW * 8 * BT), 8 * BT)],
                    osems[b])

        def drain_store(b):
            pltpu.make_async_copy(
                blk_v.at[b], out_hbm.at[pl.ds(0, BLK)], osems[b]).wait()

        def trip(c, b):
            drain_gather(b)
            pvs = [pos_v[c, pl.ds(q * LANES, LANES)] for q in range(D // LANES)]

            @pl.loop(0, BT, unroll=4)
            def _tr(j):
                for q in range(D // LANES):
                    val = rows_v[b, j, pl.ds(q * LANES, LANES)] + pvs[q]
                    plsc.store_scatter(
                        blk_v.at[b], [ivec0 + (j + q * 2048)], val)

            fire_store(c, b)
            bn = (b + 2) % NBUF

            @pl.when(jnp.logical_and(c >= 1, c + 2 < L))
            def _():
                drain_store(bn)

            @pl.when(c + 2 < L)
            def _():
                fire_gather(c + 2, bn)

        fire_gather(0, 0)
        fire_gather(1, 1)

        n_main = (L // NBUF) * NBUF

        @pl.loop(0, n_main, step=NBUF)
        def _outer(t):
            for db in range(NBUF):
                trip(t + db, db)

        for c in range(n_main, L):
            trip(c, c % NBUF)

        for c in range(L - NBUF, L):
            drain_store(c % NBUF)

    return body


def kernel(inputs, token_table, pos_table):
    B, L = inputs.shape
    V, D = token_table.shape
    info = plsc.get_sparse_core_info()
    NC, NS = info.num_cores, info.num_subcores
    NW = NC * NS
    tbl2 = jnp.pad(token_table, ((0, 0), (0, 128 - D))).reshape(2 * V, D)
    out = _build(B, L, V, D, NC, NS)(
        inputs.astype(jnp.int32), tbl2, pos_table)
    out5 = out.reshape(L, D // 8, B // BT, 8, BT)
    return out5.transpose(2, 4, 0, 1, 3).reshape(B, L, D)


# speedup vs baseline: 1.3398x; 1.3398x over previous
import functools

import jax
import jax.numpy as jnp
from jax import lax
from jax.experimental import pallas as pl
from jax.experimental.pallas import tpu as pltpu
from jax.experimental.pallas import tpu_sc as plsc

IDXW = 128
SEQ_PER_CHUNK = 2
NBUF = 3
LANES = 16


def _build(B, L, V, D, NC, NS):
    NW = NC * NS
    seqs_w = B // NW
    n_chunks = seqs_w // SEQ_PER_CHUNK
    rem = L - IDXW

    mesh = plsc.VectorSubcoreMesh(
        core_axis_name="c", subcore_axis_name="s",
        num_cores=NC, num_subcores=NS)

    @functools.partial(
        pl.kernel,
        out_type=jax.ShapeDtypeStruct((B, L, D), jnp.float32),
        mesh=mesh,
        scratch_types=[
            pltpu.VMEM((B // NW, L), jnp.int32),
            pltpu.VMEM((L, D), jnp.float32),
            pltpu.VMEM((NBUF, SEQ_PER_CHUNK, L, D), jnp.float32),
            pltpu.SemaphoreType.DMA,
            pltpu.SemaphoreType.DMA,
            pltpu.SemaphoreType.DMA,
            pltpu.SemaphoreType.DMA,
            pltpu.SemaphoreType.DMA,
            pltpu.SemaphoreType.DMA,
        ],
        compiler_params=pltpu.CompilerParams(use_tc_tiling_on_sc=False),
    )
    def body(idx_hbm, table_hbm, pos_hbm, out_hbm,
             idx_v, pos_v, rows_v, g0, g1, g2, o0, o1, o2):
        gsems = (g0, g1, g2)
        osems = (o0, o1, o2)
        wid = lax.axis_index("s") * NC + lax.axis_index("c")
        seq_base = wid * seqs_w

        pltpu.sync_copy(pos_hbm, pos_v)
        pltpu.sync_copy(idx_hbm.at[pl.ds(seq_base, seqs_w)], idx_v)

        def fire_gather(c, b):
            for s in range(SEQ_PER_CHUNK):
                row = c * SEQ_PER_CHUNK + s
                pltpu.async_copy(
                    table_hbm.at[idx_v.at[row, pl.ds(0, IDXW)]],
                    rows_v.at[b, s, pl.ds(0, IDXW)],
                    gsems[b])
                pltpu.async_copy(
                    table_hbm.at[idx_v.at[row, pl.ds(IDXW, rem)]],
                    rows_v.at[b, s, pl.ds(IDXW, rem)],
                    gsems[b])

        def drain_gather(b):
            pltpu.make_async_copy(
                table_hbm.at[pl.ds(0, L)], rows_v.at[b, 0], gsems[b]).wait()
            pltpu.make_async_copy(
                table_hbm.at[pl.ds(0, L)], rows_v.at[b, 1], gsems[b]).wait()

        def drain_store(b):
            pltpu.make_async_copy(
                rows_v.at[b], out_hbm.at[pl.ds(0, SEQ_PER_CHUNK)],
                osems[b]).wait()

        def trip(c, b):
            drain_gather(b)
            for s in range(SEQ_PER_CHUNK):
                @pl.loop(0, L, unroll=4)
                def _add(r):
                    for q in range(D // LANES):
                        plsc.addupdate(
                            rows_v.at[b, s, r, pl.ds(q * LANES, LANES)],
                            pos_v[r, pl.ds(q * LANES, LANES)])
            pltpu.async_copy(
                rows_v.at[b],
                out_hbm.at[pl.ds(seq_base + c * SEQ_PER_CHUNK, SEQ_PER_CHUNK)],
                osems[b])
            bn = (b + 2) % NBUF

            @pl.when(jnp.logical_and(c >= 1, c + 2 < n_chunks))
            def _():
                drain_store(bn)

            @pl.when(c + 2 < n_chunks)
            def _():
                fire_gather(c + 2, bn)

        fire_gather(0, 0)
        fire_gather(1, 1)

        n_main = (n_chunks // NBUF) * NBUF

        @pl.loop(0, n_main, step=NBUF)
        def _outer(t):
            for db in range(NBUF):
                trip(t + db, db)

        for c in range(n_main, n_chunks):
            trip(c, c % NBUF)

        for c in range(n_chunks - NBUF, n_chunks):
            drain_store(c % NBUF)

    return body


def kernel(inputs, token_table, pos_table):
    B, L = inputs.shape
    V, D = token_table.shape
    info = plsc.get_sparse_core_info()
    NC, NS = info.num_cores, info.num_subcores
    out = _build(B, L, V, D, NC, NS)(
        inputs.astype(jnp.int32), token_table, pos_table)
    return out
